# baseline (speedup 1.0000x reference)
import functools

import jax
import jax.numpy as jnp
from jax import lax
from jax.experimental import pallas as pl
from jax.experimental.pallas import tpu as pltpu
from jax.experimental.pallas import tpu_sc as plsc

BW = 128


@functools.cache
def _make_gather(d: int, n_b: int, n_k: int):
    info = plsc.get_sparse_core_info()
    nc, ns = info.num_cores, info.num_subcores
    nw = nc * ns
    bs = n_b // nw
    n_w = bs // BW
    n_u = n_k * n_w
    mesh = plsc.VectorSubcoreMesh(core_axis_name="c", subcore_axis_name="s")

    @functools.partial(
        pl.kernel,
        mesh=mesh,
        out_type=jax.ShapeDtypeStruct((n_k, d, n_b), jnp.float32),
        scratch_types=[
            pltpu.VMEM((n_k, bs), jnp.int32),
            [pltpu.VMEM((d, BW), jnp.float32)] * 2,
            [pltpu.SemaphoreType.DMA] * 2,
            [pltpu.SemaphoreType.DMA] * 2,
        ],
    )
    def gather_kernel(table_hbm, it_hbm, out_hbm, idx_v, bufs, gsems, ssems):
        wid = lax.axis_index("s") * nc + lax.axis_index("c")
        b0 = wid * bs
        pltpu.sync_copy(it_hbm.at[:, pl.ds(b0, bs)], idx_v)

        def issue(p, u):
            k = u // n_w
            woff = (u % n_w) * BW

            def vec_body(v, carry):
                vec = idx_v[k, pl.ds(woff + v * 16, 16)]
                for lane in range(16):
                    pltpu.async_copy(
                        table_hbm.at[vec[lane]],
                        bufs[p].at[:, v * 16 + lane],
                        gsems[p],
                    )
                return carry

            lax.fori_loop(0, BW // 16, vec_body, 0)

        def drain_gather(p):
            pltpu.make_async_copy(
                out_hbm.at[0, :, pl.ds(0, BW)], bufs[p], gsems[p]
            ).wait()

        def out_slice(u):
            k = u // n_w
            woff = (u % n_w) * BW
            return out_hbm.at[k, :, pl.ds(b0 + woff, BW)]

        def store(p, u):
            pltpu.async_copy(bufs[p], out_slice(u), ssems[p])

        def drain_store(p, u):
            pltpu.make_async_copy(bufs[p], out_slice(u), ssems[p]).wait()

        issue(0, 0)
        issue(1, 1)

        def group_body(g, carry):
            u0 = 2 * g
            for p in range(2):
                drain_gather(p)
                store(p, u0 + p)
            for p in range(2):
                drain_store(p, u0 + p)
                issue(p, u0 + 2 + p)
            return carry

        lax.fori_loop(0, n_u // 2 - 1, group_body, 0)

        u0 = n_u - 2
        for p in range(2):
            drain_gather(p)
            store(p, u0 + p)
        for p in range(2):
            drain_store(p, u0 + p)

    return gather_kernel


def kernel(vars_, index):
    n_b, n_k = index.shape
    d = vars_.shape[1]
    it = index.T.astype(jnp.int32)
    out = _make_gather(d, n_b, n_k)(vars_, it)
    return out.transpose(2, 0, 1)


# speedup vs baseline: 4.6844x; 4.6844x over previous
import functools

import jax
import jax.numpy as jnp
from jax import lax
from jax.experimental import pallas as pl
from jax.experimental.pallas import tpu as pltpu
from jax.experimental.pallas import tpu_sc as plsc

ROWS_PER_BLOCK = 26
BLOCKS_PER_CHUNK = 4
CHUNK = ROWS_PER_BLOCK * BLOCKS_PER_CHUNK
NBUF = 8


@functools.cache
def _make_gather(n_b: int, n_k: int, d: int):
    info = plsc.get_sparse_core_info()
    nc, ns = info.num_cores, info.num_subcores
    nw = nc * ns
    n_chunks = (n_b // BLOCKS_PER_CHUNK)
    chunks_per_w = n_chunks // nw
    n_groups = chunks_per_w // NBUF
    mesh = plsc.VectorSubcoreMesh(core_axis_name="c", subcore_axis_name="s")

    @functools.partial(
        pl.kernel,
        mesh=mesh,
        out_type=jax.ShapeDtypeStruct((n_b, n_k, d), jnp.float32),
        scratch_types=[
            pltpu.VMEM((chunks_per_w, CHUNK), jnp.int32),
            pltpu.VMEM((NBUF, CHUNK, d), jnp.float32),
            [pltpu.SemaphoreType.DMA] * NBUF,
            [pltpu.SemaphoreType.DMA] * NBUF,
        ],
    )
    def gather_kernel(table_hbm, idx_hbm, out_hbm, idx_v, bufs, gsems, ssems):
        wid = lax.axis_index("s") * nc + lax.axis_index("c")
        base = wid * chunks_per_w
        pltpu.sync_copy(idx_hbm.at[pl.ds(base, chunks_per_w)], idx_v)

        def stage(b, j):
            del b, j

        _windows = [(i * 16, 0) for i in range(CHUNK // 16)]
        if CHUNK % 16:
            _windows.append((CHUNK - 16, 16 - CHUNK % 16))

        def gather_issue(b, j):
            for off, lo in _windows:
                vec = idx_v[j, pl.ds(off, 16)]
                for lane in range(lo, 16):
                    pltpu.async_copy(
                        table_hbm.at[vec[lane]], bufs.at[b, off + lane], gsems[b]
                    )

        def drain_gather(b):
            pltpu.make_async_copy(
                table_hbm.at[pl.ds(0, CHUNK)], bufs.at[b], gsems[b]
            ).wait()

        def store(b, j):
            c = base + j
            for i in range(BLOCKS_PER_CHUNK):
                pltpu.async_copy(
                    bufs.at[b, pl.ds(i * ROWS_PER_BLOCK, ROWS_PER_BLOCK)],
                    out_hbm.at[c * BLOCKS_PER_CHUNK + i],
                    ssems[b],
                )

        def drain_store(b, j):
            c = base + j
            for i in range(BLOCKS_PER_CHUNK):
                pltpu.make_async_copy(
                    bufs.at[b, pl.ds(i * ROWS_PER_BLOCK, ROWS_PER_BLOCK)],
                    out_hbm.at[c * BLOCKS_PER_CHUNK + i],
                    ssems[b],
                ).wait()

        for b in range(NBUF):
            stage(b, b)
            gather_issue(b, b)

        def group_body(g, carry):
            j0 = g * NBUF
            for b in range(NBUF):
                drain_gather(b)
                store(b, j0 + b)
            for b in range(NBUF):
                drain_store(b, j0 + b)
                stage(b, j0 + NBUF + b)
                gather_issue(b, j0 + NBUF + b)
            return carry

        lax.fori_loop(0, n_groups - 1, group_body, 0)

        j0 = (n_groups - 1) * NBUF
        for b in range(NBUF):
            drain_gather(b)
            store(b, j0 + b)
        for b in range(NBUF):
            drain_store(b, j0 + b)

    return gather_kernel


def kernel(vars_, index):
    n_b, n_k = index.shape
    d = vars_.shape[1]
    idx = index.reshape(n_b // BLOCKS_PER_CHUNK, CHUNK).astype(jnp.int32)
    return _make_gather(n_b, n_k, d)(vars_, idx)
